# baseline (speedup 1.0000x reference)
import jax
import jax.numpy as jnp
from jax import lax
from jax.experimental import pallas as pl
from jax.experimental.pallas import tpu as pltpu
from jax.experimental.pallas import tpu_sc as plsc

_D_IN = 4096
_D_H = 2048
_NC = 2
_NS = 16
_L = 16
_NW = _NC * _NS
_ROWS_W = _D_IN // _NW
_RCH = 16
_NRCH = _ROWS_W // _RCH
_SW = 8
_NSTRIP = _D_H // (_SW * _L)
_RG = 8


def _rbf16(x):
    u = plsc.bitcast(x, jnp.int32)
    return plsc.bitcast((u + 0x8000) & jnp.int32(-65536), jnp.float32)


def _rbf16_inplace(ref, n):
    def body(k, _):
        ref[pl.ds(k * _L, _L)] = _rbf16(ref[pl.ds(k * _L, _L)])
        return 0

    lax.fori_loop(0, n // _L, body, 0)


def _splat(v, lane):
    idx = jnp.full((_L,), lane, dtype=jnp.int32)
    return v.at[idx].get(mode="promise_in_bounds")


def _sc_body(w_hbm, a_hbm, b_hbm, ws_hbm, be_hbm, out_hbm,
             buf0, buf1, a_v, b_v, ws1_v, ws2_v, be_v, pv_v,
             sem0, sem1, sem2):
    cid = lax.axis_index("c")
    sid = lax.axis_index("s")
    wid = sid * _NC + cid
    row0 = wid * _ROWS_W
    bufs = [buf0, buf1]
    sems = [sem0, sem1]
    handles = [
        pltpu.async_copy(
            w_hbm.at[pl.ds(row0 + c * _RCH, _RCH), :], bufs[c], sems[c])
        for c in range(2)
    ]
    small = [
        pltpu.async_copy(a_hbm.at[pl.ds(row0, _ROWS_W)], a_v, sem2),
        pltpu.async_copy(b_hbm.at[pl.ds(row0, _ROWS_W)], b_v, sem2),
        pltpu.async_copy(ws_hbm.at[pl.ds(0, _D_H)], ws1_v, sem2),
        pltpu.async_copy(ws_hbm.at[pl.ds(_D_H, _D_H)], ws2_v, sem2),
        pltpu.async_copy(be_hbm, be_v, sem2),
    ]
    for h in small:
        h.wait()
    _rbf16_inplace(a_v, _ROWS_W)
    _rbf16_inplace(b_v, _ROWS_W)

    zero = jnp.zeros((_L,), jnp.float32)
    tots = (zero, zero)
    for c in range(_NRCH):
        handles[c % 2].wait()
        buf = bufs[c % 2]
        av = a_v[pl.ds(c * _RCH, _L)]
        bv = b_v[pl.ds(c * _RCH, _L)]

        def strip_body(t, tt, buf=buf, av=av, bv=bv):
            t1, t2 = tt
            col0 = t * (_SW * _L)
            zero_ = jnp.zeros((_L,), jnp.float32)

            def row_body(g, accs, buf=buf):
                a1 = list(accs[:_SW])
                a2 = list(accs[_SW:])
                for i in range(_RG):
                    lane = g * _RG + i
                    ai = _splat(av, lane)
                    bi = _splat(bv, lane)
                    for u in range(_SW):
                        w = _rbf16(buf[lane, pl.ds(col0 + u * _L, _L)])
                        a1[u] = a1[u] + ai * w
                        a2[u] = a2[u] + bi * w
                return tuple(a1) + tuple(a2)

            accs = lax.fori_loop(0, _RCH // _RG, row_body,
                                 (zero_,) * (2 * _SW))
            for u in range(_SW):
                t1 = t1 + accs[u] * ws1_v[pl.ds(col0 + u * _L, _L)]
                t2 = t2 + accs[_SW + u] * ws2_v[pl.ds(col0 + u * _L, _L)]
            return (t1, t2)

        tots = lax.fori_loop(0, _NSTRIP, strip_body, tots)
        if c + 2 < _NRCH:
            handles[c % 2] = pltpu.async_copy(
                w_hbm.at[pl.ds(row0 + (c + 2) * _RCH, _RCH), :],
                bufs[c % 2], sems[c % 2])

    pv_v[...] = tots[0] + tots[1]

    @pl.when(wid == 0)
    def _():
        def bias_body(k, bv_):
            c0 = k * _L
            return bv_ + be_v[pl.ds(c0, _L)] * (
                ws1_v[pl.ds(c0, _L)] + ws2_v[pl.ds(c0, _L)])

        bias_v = lax.fori_loop(0, _D_H // _L, bias_body, zero)
        pv_v[...] = pv_v[...] + bias_v

    pltpu.sync_copy(pv_v, out_hbm.at[wid])


def kernel(a, b, W_embed, b_embed, W_scorer, b_scorer):
    mesh = plsc.VectorSubcoreMesh(core_axis_name="c", subcore_axis_name="s")
    run = pl.kernel(
        _sc_body,
        mesh=mesh,
        compiler_params=pltpu.CompilerParams(needs_layout_passes=False),
        out_type=jax.ShapeDtypeStruct((_NW, _L), jnp.float32),
        scratch_types=[
            pltpu.VMEM((_RCH, _D_H), jnp.float32),
            pltpu.VMEM((_RCH, _D_H), jnp.float32),
            pltpu.VMEM((_ROWS_W,), jnp.float32),
            pltpu.VMEM((_ROWS_W,), jnp.float32),
            pltpu.VMEM((_D_H,), jnp.float32),
            pltpu.VMEM((_D_H,), jnp.float32),
            pltpu.VMEM((_D_H,), jnp.float32),
            pltpu.VMEM((_L,), jnp.float32),
            pltpu.SemaphoreType.DMA,
            pltpu.SemaphoreType.DMA,
            pltpu.SemaphoreType.DMA,
        ],
    )
    parts = run(W_embed, a, b, W_scorer.reshape(-1), b_embed)
    return jnp.sum(parts) + b_scorer[0]
